# baseline (speedup 1.0000x reference)
import functools

import jax
import jax.numpy as jnp
from jax import lax
from jax.experimental import pallas as pl
from jax.experimental.pallas import tpu as pltpu
from jax.experimental.pallas import tpu_sc as plsc

_EPS = 1e-5
_LANES = 16


def _sc_geometry():
    try:
        info = plsc.get_sparse_core_info()
        return info.num_cores, info.num_subcores
    except Exception:
        return 2, 16


def _build_table(tok, pos, seg, gamma, beta, seq_len):
    V, D = tok.shape
    NS = seg.shape[0]

    def body(tok_ref, pos_ref, seg_ref, gam_ref, bet_ref, out_ref):
        tokv = tok_ref[...]
        posv = pos_ref[:seq_len, :]
        segv = seg_ref[...]
        e = (tokv[:, None, None, :] + posv[None, None, :, :]) + segv[None, :, None, :]
        mean = jnp.mean(e, axis=-1, keepdims=True)
        c = e - mean
        var = jnp.mean(c * c, axis=-1, keepdims=True)
        y = c * lax.rsqrt(var + _EPS)
        y = y * gam_ref[...] + bet_ref[...]
        out_ref[...] = y.reshape(V * NS, seq_len, D)

    out = pl.pallas_call(
        body,
        out_shape=jax.ShapeDtypeStruct((V * NS, seq_len, D), jnp.float32),
    )(tok, pos, seg, gamma.reshape(1, -1), beta.reshape(1, -1))
    return out.reshape(V * NS * seq_len, D)


def _sc_gather(lnt, xf, sf, seq_len, nseg):
    T, D = lnt.shape
    BT = xf.shape[0]
    NC, NSUB = _sc_geometry()
    NW = NC * NSUB
    assert BT % (NW * _LANES) == 0
    per_w = BT // NW
    CH = 64
    NBUF = 4
    assert per_w % (NBUF * CH) == 0
    nch = per_w // CH

    mesh = plsc.VectorSubcoreMesh(
        core_axis_name="c", subcore_axis_name="s",
        num_cores=NC, num_subcores=NSUB)

    @functools.partial(
        pl.kernel,
        out_type=jax.ShapeDtypeStruct((BT, D), jnp.int32),
        mesh=mesh,
        scratch_types=[
            pltpu.VMEM((per_w,), jnp.int32),
            pltpu.VMEM((per_w,), jnp.int32),
            pltpu.VMEM((per_w,), jnp.int32),
            [pltpu.VMEM((CH, D), jnp.int32) for _ in range(NBUF)],
            [pltpu.SemaphoreType.DMA for _ in range(NBUF)],
            [pltpu.SemaphoreType.DMA for _ in range(NBUF)],
        ],
    )
    def k(x_hbm, s_hbm, lnt_hbm, out_hbm, xv, sv, iv, bufs, gsems, tsems):
        sid = lax.axis_index("s")
        wid = sid * NC + lax.axis_index("c")
        base = wid * per_w

        pltpu.sync_copy(x_hbm.at[pl.ds(base, per_w)], xv)
        pltpu.sync_copy(s_hbm.at[pl.ds(base, per_w)], sv)

        lane = lax.iota(jnp.int32, _LANES)

        @pl.loop(0, per_w // _LANES)
        def _(i):
            off = i * _LANES
            xi = xv[pl.ds(off, _LANES)]
            si = sv[pl.ds(off, _LANES)]
            j = lax.rem(base + off + lane, seq_len)
            iv[pl.ds(off, _LANES)] = (xi * nseg + si) * seq_len + j

        def g_start(c, buf, sem):
            pltpu.async_copy(lnt_hbm.at[iv.at[pl.ds(c * CH, CH)]], buf, sem)

        def g_wait(buf, sem):
            pltpu.make_async_copy(lnt_hbm.at[pl.ds(0, CH)], buf, sem).wait()

        def s_start(c, buf, sem):
            return pltpu.async_copy(buf, out_hbm.at[pl.ds(base + c * CH, CH)],
                                    sem)

        for b in range(NBUF):
            g_start(b, bufs[b], gsems[b])

        @pl.loop(0, nch // NBUF)
        def _(t):
            c0 = t * NBUF
            sts = []
            for b in range(NBUF):
                g_wait(bufs[b], gsems[b])
                sts.append(s_start(c0 + b, bufs[b], tsems[b]))
            for b in range(NBUF):
                sts[b].wait()

                @pl.when(c0 + b + NBUF < nch)
                def _(b=b):
                    g_start(c0 + b + NBUF, bufs[b], gsems[b])

    return k(xf, sf, lnt)


def _relayout(o2, B, S, D):
    BS = 64

    def body(in_ref, out_ref):
        for k in range(BS):
            out_ref[k] = in_ref[k * S:(k + 1) * S].astype(jnp.float32)

    return pl.pallas_call(
        body,
        grid=(B // BS,),
        in_specs=[pl.BlockSpec((BS * S, D), lambda i: (i, 0))],
        out_specs=pl.BlockSpec((BS, S, D), lambda i: (i, 0, 0)),
        out_shape=jax.ShapeDtypeStruct((B, S, D), jnp.float32),
    )(o2)


def kernel(x, seg, tok_table, pos_table, seg_table, gamma, beta):
    B, S = x.shape
    NS = seg_table.shape[0]
    D = tok_table.shape[1]
    lnt = _build_table(tok_table, pos_table, seg_table, gamma, beta, S)
    xf = x.reshape(-1).astype(jnp.int32)
    sf = seg.reshape(-1).astype(jnp.int32)
    lnt16 = lnt.astype(jnp.bfloat16)
    lnt32 = lax.bitcast_convert_type(lnt16.reshape(-1, D // 2, 2), jnp.int32)
    out32 = _sc_gather(lnt32, xf, sf, S, NS)
    out16 = lax.bitcast_convert_type(out32, jnp.bfloat16).reshape(B * S, D)
    return _relayout(out16, B, S, D)


# speedup vs baseline: 2.9652x; 2.9652x over previous
import functools

import jax
import jax.numpy as jnp
from jax import lax
from jax.experimental import pallas as pl
from jax.experimental.pallas import tpu as pltpu
from jax.experimental.pallas import tpu_sc as plsc

_EPS = 1e-5
_LANES = 16


def _sc_geometry():
    try:
        info = plsc.get_sparse_core_info()
        return info.num_cores, info.num_subcores
    except Exception:
        return 2, 16


def _build_table(tok, pos, seg, gamma, beta, seq_len):
    V, D = tok.shape
    NS = seg.shape[0]

    def body(tok_ref, pos_ref, seg_ref, gam_ref, bet_ref, out_ref):
        tokv = tok_ref[...]
        posv = pos_ref[:seq_len, :]
        segv = seg_ref[...]
        e = (tokv[:, None, None, :] + posv[None, None, :, :]) + segv[None, :, None, :]
        mean = jnp.mean(e, axis=-1, keepdims=True)
        c = e - mean
        var = jnp.mean(c * c, axis=-1, keepdims=True)
        y = c * lax.rsqrt(var + _EPS)
        y = y * gam_ref[...] + bet_ref[...]
        out_ref[...] = y.reshape(V * NS, seq_len, D)

    out = pl.pallas_call(
        body,
        out_shape=jax.ShapeDtypeStruct((V * NS, seq_len, D), jnp.float32),
    )(tok, pos, seg, gamma.reshape(1, -1), beta.reshape(1, -1))
    return out.reshape(V * NS * seq_len, D)


def _sc_gather(lnt, xf, sf, seq_len, nseg):
    T, D = lnt.shape
    BT = xf.shape[0]
    NC, NSUB = _sc_geometry()
    NW = NC * NSUB
    assert BT % (NW * _LANES) == 0
    per_w = BT // NW
    CH = 64
    NBUF = 4
    assert per_w % (NBUF * CH) == 0
    nch = per_w // CH

    mesh = plsc.VectorSubcoreMesh(
        core_axis_name="c", subcore_axis_name="s",
        num_cores=NC, num_subcores=NSUB)

    @functools.partial(
        pl.kernel,
        out_type=jax.ShapeDtypeStruct((BT, D), jnp.int32),
        mesh=mesh,
        scratch_types=[
            pltpu.VMEM((per_w,), jnp.int32),
            pltpu.VMEM((per_w,), jnp.int32),
            pltpu.VMEM((per_w,), jnp.int32),
            [pltpu.VMEM((CH, D), jnp.int32) for _ in range(NBUF)],
            [pltpu.SemaphoreType.DMA for _ in range(NBUF)],
            [pltpu.SemaphoreType.DMA for _ in range(NBUF)],
        ],
    )
    def k(x_hbm, s_hbm, lnt_hbm, out_hbm, xv, sv, iv, bufs, gsems, tsems):
        sid = lax.axis_index("s")
        wid = sid * NC + lax.axis_index("c")
        base = wid * per_w

        pltpu.sync_copy(x_hbm.at[pl.ds(base, per_w)], xv)
        pltpu.sync_copy(s_hbm.at[pl.ds(base, per_w)], sv)

        lane = lax.iota(jnp.int32, _LANES)

        @pl.loop(0, per_w // _LANES)
        def _(i):
            off = i * _LANES
            xi = xv[pl.ds(off, _LANES)]
            si = sv[pl.ds(off, _LANES)]
            j = lax.rem(base + off + lane, seq_len)
            iv[pl.ds(off, _LANES)] = (xi * nseg + si) * seq_len + j

        def g_start(c, buf, sem):
            pltpu.async_copy(lnt_hbm.at[iv.at[pl.ds(c * CH, CH)]], buf, sem)

        def g_wait(buf, sem):
            pltpu.make_async_copy(lnt_hbm.at[pl.ds(0, CH)], buf, sem).wait()

        def s_start(c, buf, sem):
            return pltpu.async_copy(buf, out_hbm.at[pl.ds(base + c * CH, CH)],
                                    sem)

        for b in range(NBUF):
            g_start(b, bufs[b], gsems[b])

        @pl.loop(0, nch // NBUF)
        def _(t):
            c0 = t * NBUF
            sts = []
            for b in range(NBUF):
                g_wait(bufs[b], gsems[b])
                sts.append(s_start(c0 + b, bufs[b], tsems[b]))
            for b in range(NBUF):
                sts[b].wait()

                @pl.when(c0 + b + NBUF < nch)
                def _(b=b):
                    g_start(c0 + b + NBUF, bufs[b], gsems[b])

    return k(xf, sf, lnt)


def _relayout(o2, B, S, D):
    BS = 64
    H = D // 2

    def body(in_ref, out_ref):
        for k in range(BS):
            xi = in_ref[k * S:(k + 1) * S]
            out_ref[k, :, :H] = lax.bitcast_convert_type(
                xi << 16, jnp.float32)
            out_ref[k, :, H:] = lax.bitcast_convert_type(
                (xi >> 16) << 16, jnp.float32)

    return pl.pallas_call(
        body,
        grid=(B // BS,),
        in_specs=[pl.BlockSpec((BS * S, H), lambda i: (i, 0))],
        out_specs=pl.BlockSpec((BS, S, D), lambda i: (i, 0, 0)),
        out_shape=jax.ShapeDtypeStruct((B, S, D), jnp.float32),
    )(o2)


def kernel(x, seg, tok_table, pos_table, seg_table, gamma, beta):
    B, S = x.shape
    NS = seg_table.shape[0]
    D = tok_table.shape[1]
    lnt = _build_table(tok_table, pos_table, seg_table, gamma, beta, S)
    xf = x.reshape(-1).astype(jnp.int32)
    sf = seg.reshape(-1).astype(jnp.int32)
    lnt16 = lnt.astype(jnp.bfloat16)
    a = lax.bitcast_convert_type(lnt16[:, :D // 2], jnp.uint16).astype(jnp.uint32)
    b = lax.bitcast_convert_type(lnt16[:, D // 2:], jnp.uint16).astype(jnp.uint32)
    packed = lax.bitcast_convert_type((b << 16) | a, jnp.int32)
    out32 = _sc_gather(packed, xf, sf, S, NS)
    return _relayout(out32, B, S, D)


# speedup vs baseline: 3.1296x; 1.0555x over previous
import functools

import jax
import jax.numpy as jnp
from jax import lax
from jax.experimental import pallas as pl
from jax.experimental.pallas import tpu as pltpu
from jax.experimental.pallas import tpu_sc as plsc

_EPS = 1e-5
_LANES = 16


def _sc_geometry():
    try:
        info = plsc.get_sparse_core_info()
        return info.num_cores, info.num_subcores
    except Exception:
        return 2, 16


def _build_table(tok, pos, seg, gamma, beta, seq_len):
    V, D = tok.shape
    NS = seg.shape[0]

    def body(tok_ref, pos_ref, seg_ref, gam_ref, bet_ref, out_ref):
        tokv = tok_ref[...]
        posv = pos_ref[:seq_len, :]
        segv = seg_ref[...]
        e = (tokv[:, None, None, :] + posv[None, None, :, :]) + segv[None, :, None, :]
        mean = jnp.mean(e, axis=-1, keepdims=True)
        c = e - mean
        var = jnp.mean(c * c, axis=-1, keepdims=True)
        y = c * lax.rsqrt(var + _EPS)
        y = y * gam_ref[...] + bet_ref[...]
        out_ref[...] = y.reshape(V * NS, seq_len, D)

    out = pl.pallas_call(
        body,
        out_shape=jax.ShapeDtypeStruct((V * NS, seq_len, D), jnp.float32),
    )(tok, pos, seg, gamma.reshape(1, -1), beta.reshape(1, -1))
    return out.reshape(V * NS * seq_len, D)


def _sc_gather(lnt, xf, sf, seq_len, nseg):
    T, D = lnt.shape
    BT = xf.shape[0]
    NC, NSUB = _sc_geometry()
    NW = NC * NSUB
    assert BT % (NW * _LANES) == 0
    per_w = BT // NW
    CH = 32
    NBUF = 4
    assert per_w % (NBUF * CH) == 0
    nch = per_w // CH

    mesh = plsc.VectorSubcoreMesh(
        core_axis_name="c", subcore_axis_name="s",
        num_cores=NC, num_subcores=NSUB)

    @functools.partial(
        pl.kernel,
        out_type=jax.ShapeDtypeStruct((BT, D), jnp.int32),
        mesh=mesh,
        scratch_types=[
            pltpu.VMEM((per_w,), jnp.int32),
            pltpu.VMEM((per_w,), jnp.int32),
            pltpu.VMEM((per_w,), jnp.int32),
            [pltpu.VMEM((CH, D), jnp.int32) for _ in range(NBUF)],
            [pltpu.SemaphoreType.DMA for _ in range(NBUF)],
            [pltpu.SemaphoreType.DMA for _ in range(NBUF)],
        ],
    )
    def k(x_hbm, s_hbm, lnt_hbm, out_hbm, xv, sv, iv, bufs, gsems, tsems):
        sid = lax.axis_index("s")
        wid = sid * NC + lax.axis_index("c")
        base = wid * per_w

        pltpu.sync_copy(x_hbm.at[pl.ds(base, per_w)], xv)
        pltpu.sync_copy(s_hbm.at[pl.ds(base, per_w)], sv)

        lane = lax.iota(jnp.int32, _LANES)

        @pl.loop(0, per_w // _LANES)
        def _(i):
            off = i * _LANES
            xi = xv[pl.ds(off, _LANES)]
            si = sv[pl.ds(off, _LANES)]
            j = lax.rem(base + off + lane, seq_len)
            iv[pl.ds(off, _LANES)] = (xi * nseg + si) * seq_len + j

        def g_start(c, buf, sem):
            pltpu.async_copy(lnt_hbm.at[iv.at[pl.ds(c * CH, CH)]], buf, sem)

        def g_wait(buf, sem):
            pltpu.make_async_copy(lnt_hbm.at[pl.ds(0, CH)], buf, sem).wait()

        def s_start(c, buf, sem):
            return pltpu.async_copy(buf, out_hbm.at[pl.ds(base + c * CH, CH)],
                                    sem)

        for b in range(NBUF):
            g_start(b, bufs[b], gsems[b])

        @pl.loop(0, nch // NBUF)
        def _(t):
            c0 = t * NBUF
            sts = []
            for b in range(NBUF):
                g_wait(bufs[b], gsems[b])
                sts.append(s_start(c0 + b, bufs[b], tsems[b]))
            for b in range(NBUF):
                sts[b].wait()

                @pl.when(c0 + b + NBUF < nch)
                def _(b=b):
                    g_start(c0 + b + NBUF, bufs[b], gsems[b])

    return k(xf, sf, lnt)


def _relayout_chunk(o2, prev, chunk_blk0, B, S, D):
    BS = 64
    H = D // 2
    BC = o2.shape[0] // S

    def body(in_ref, *rest):
        out_ref = rest[-1]
        for k in range(BS):
            xi = in_ref[k * S:(k + 1) * S]
            out_ref[k, :, :H] = lax.bitcast_convert_type(
                xi << 16, jnp.float32)
            out_ref[k, :, H:] = lax.bitcast_convert_type(
                (xi >> 16) << 16, jnp.float32)

    in_specs = [pl.BlockSpec((BS * S, H), lambda i: (i, 0))]
    args = [o2]
    aliases = {}
    if prev is not None:
        in_specs.append(pl.BlockSpec(memory_space=pl.ANY))
        args.append(prev)
        aliases = {1: 0}
    return pl.pallas_call(
        body,
        grid=(BC // BS,),
        in_specs=in_specs,
        out_specs=pl.BlockSpec((BS, S, D), lambda i: (chunk_blk0 + i, 0, 0)),
        out_shape=jax.ShapeDtypeStruct((B, S, D), jnp.float32),
        input_output_aliases=aliases,
    )(*args)


def kernel(x, seg, tok_table, pos_table, seg_table, gamma, beta):
    B, S = x.shape
    NS = seg_table.shape[0]
    D = tok_table.shape[1]
    lnt = _build_table(tok_table, pos_table, seg_table, gamma, beta, S)
    xf = x.reshape(-1).astype(jnp.int32)
    sf = seg.reshape(-1).astype(jnp.int32)
    lnt16 = lnt.astype(jnp.bfloat16)
    a = lax.bitcast_convert_type(lnt16[:, :D // 2], jnp.uint16).astype(jnp.uint32)
    b = lax.bitcast_convert_type(lnt16[:, D // 2:], jnp.uint16).astype(jnp.uint32)
    packed = lax.bitcast_convert_type((b << 16) | a, jnp.int32)
    NCK = 4
    BC = B // NCK
    RC = BC * S
    out = None
    for kchunk in range(NCK):
        o32 = _sc_gather(packed, xf[kchunk * RC:(kchunk + 1) * RC],
                         sf[kchunk * RC:(kchunk + 1) * RC], S, NS)
        out = _relayout_chunk(o32, out, kchunk * (BC // 64), B, S, D)
    return out


# speedup vs baseline: 3.1399x; 1.0033x over previous
import functools

import jax
import jax.numpy as jnp
from jax import lax
from jax.experimental import pallas as pl
from jax.experimental.pallas import tpu as pltpu
from jax.experimental.pallas import tpu_sc as plsc

_EPS = 1e-5
_LANES = 16


def _sc_geometry():
    try:
        info = plsc.get_sparse_core_info()
        return info.num_cores, info.num_subcores
    except Exception:
        return 2, 16


def _build_table(tok, pos, seg, gamma, beta, seq_len):
    V, D = tok.shape
    NS = seg.shape[0]

    def body(tok_ref, pos_ref, seg_ref, gam_ref, bet_ref, out_ref):
        tokv = tok_ref[...]
        posv = pos_ref[:seq_len, :]
        segv = seg_ref[...]
        e = (tokv[:, None, None, :] + posv[None, None, :, :]) + segv[None, :, None, :]
        mean = jnp.mean(e, axis=-1, keepdims=True)
        c = e - mean
        var = jnp.mean(c * c, axis=-1, keepdims=True)
        y = c * lax.rsqrt(var + _EPS)
        y = y * gam_ref[...] + bet_ref[...]
        out_ref[...] = y.reshape(V * NS, seq_len, D)

    out = pl.pallas_call(
        body,
        out_shape=jax.ShapeDtypeStruct((V * NS, seq_len, D), jnp.float32),
    )(tok, pos, seg, gamma.reshape(1, -1), beta.reshape(1, -1))
    return out.reshape(V * NS * seq_len, D)


def _sc_gather(lnt, xf, sf, seq_len, nseg):
    T, D = lnt.shape
    BT = xf.shape[0]
    NC, NSUB = _sc_geometry()
    NW = NC * NSUB
    assert BT % (NW * _LANES) == 0
    per_w = BT // NW
    CH = 32
    NBUF = 4
    assert per_w % (NBUF * CH) == 0
    nch = per_w // CH

    mesh = plsc.VectorSubcoreMesh(
        core_axis_name="c", subcore_axis_name="s",
        num_cores=NC, num_subcores=NSUB)

    @functools.partial(
        pl.kernel,
        out_type=jax.ShapeDtypeStruct((BT, D), jnp.int32),
        mesh=mesh,
        scratch_types=[
            pltpu.VMEM((per_w,), jnp.int32),
            pltpu.VMEM((per_w,), jnp.int32),
            pltpu.VMEM((per_w,), jnp.int32),
            [pltpu.VMEM((CH, D), jnp.int32) for _ in range(NBUF)],
            [pltpu.SemaphoreType.DMA for _ in range(NBUF)],
            [pltpu.SemaphoreType.DMA for _ in range(NBUF)],
        ],
    )
    def k(x_hbm, s_hbm, lnt_hbm, out_hbm, xv, sv, iv, bufs, gsems, tsems):
        sid = lax.axis_index("s")
        wid = sid * NC + lax.axis_index("c")
        base = wid * per_w

        pltpu.sync_copy(x_hbm.at[pl.ds(base, per_w)], xv)
        pltpu.sync_copy(s_hbm.at[pl.ds(base, per_w)], sv)

        lane = lax.iota(jnp.int32, _LANES)

        @pl.loop(0, per_w // _LANES)
        def _(i):
            off = i * _LANES
            xi = xv[pl.ds(off, _LANES)]
            si = sv[pl.ds(off, _LANES)]
            j = lax.rem(base + off + lane, seq_len)
            iv[pl.ds(off, _LANES)] = (xi * nseg + si) * seq_len + j

        def g_start(c, buf, sem):
            pltpu.async_copy(lnt_hbm.at[iv.at[pl.ds(c * CH, CH)]], buf, sem)

        def g_wait(buf, sem):
            pltpu.make_async_copy(lnt_hbm.at[pl.ds(0, CH)], buf, sem).wait()

        def s_start(c, buf, sem):
            return pltpu.async_copy(buf, out_hbm.at[pl.ds(base + c * CH, CH)],
                                    sem)

        for b in range(NBUF):
            g_start(b, bufs[b], gsems[b])

        @pl.loop(0, nch // NBUF)
        def _(t):
            c0 = t * NBUF
            sts = []
            for b in range(NBUF):
                g_wait(bufs[b], gsems[b])
                sts.append(s_start(c0 + b, bufs[b], tsems[b]))
            for b in range(NBUF):
                sts[b].wait()

                @pl.when(c0 + b + NBUF < nch)
                def _(b=b):
                    g_start(c0 + b + NBUF, bufs[b], gsems[b])

    return k(xf, sf, lnt)


def _relayout_chunk(o2, prev, chunk_blk0, B, S, D):
    BS = 128
    H = D // 2
    BC = o2.shape[0] // S

    def body(in_ref, *rest):
        out_ref = rest[-1]
        for k in range(BS):
            xi = in_ref[k * S:(k + 1) * S]
            out_ref[k, :, :H] = lax.bitcast_convert_type(
                xi << 16, jnp.float32)
            out_ref[k, :, H:] = lax.bitcast_convert_type(
                (xi >> 16) << 16, jnp.float32)

    in_specs = [pl.BlockSpec((BS * S, H), lambda i: (i, 0))]
    args = [o2]
    aliases = {}
    if prev is not None:
        in_specs.append(pl.BlockSpec(memory_space=pl.ANY))
        args.append(prev)
        aliases = {1: 0}
    return pl.pallas_call(
        body,
        grid=(BC // BS,),
        in_specs=in_specs,
        out_specs=pl.BlockSpec((BS, S, D), lambda i: (chunk_blk0 + i, 0, 0)),
        out_shape=jax.ShapeDtypeStruct((B, S, D), jnp.float32),
        input_output_aliases=aliases,
    )(*args)


def kernel(x, seg, tok_table, pos_table, seg_table, gamma, beta):
    B, S = x.shape
    NS = seg_table.shape[0]
    D = tok_table.shape[1]
    lnt = _build_table(tok_table, pos_table, seg_table, gamma, beta, S)
    xf = x.reshape(-1).astype(jnp.int32)
    sf = seg.reshape(-1).astype(jnp.int32)
    lnt16 = lnt.astype(jnp.bfloat16)
    a = lax.bitcast_convert_type(lnt16[:, :D // 2], jnp.uint16).astype(jnp.uint32)
    b = lax.bitcast_convert_type(lnt16[:, D // 2:], jnp.uint16).astype(jnp.uint32)
    packed = lax.bitcast_convert_type((b << 16) | a, jnp.int32)
    NCK = 4
    BC = B // NCK
    RC = BC * S
    out = None
    for kchunk in range(NCK):
        o32 = _sc_gather(packed, xf[kchunk * RC:(kchunk + 1) * RC],
                         sf[kchunk * RC:(kchunk + 1) * RC], S, NS)
        out = _relayout_chunk(o32, out, kchunk * (BC // 64), B, S, D)
    return out
